# Initial kernel scaffold; baseline (speedup 1.0000x reference)
import jax
import jax.numpy as jnp
from jax.experimental import pallas as pl


def kernel(feats, segment_ids, W, b):
    raise NotImplementedError("write your pallas kernel here")


# speedup vs baseline: 3.3545x; 3.3545x over previous
import functools

import jax
import jax.numpy as jnp
from jax import lax
from jax.experimental import pallas as pl
from jax.experimental.pallas import tpu as pltpu
from jax.experimental.pallas import tpu_sc as plsc

NUM_SEGMENTS = 512
FEATS = 128
LANES = 16
NCHUNK = FEATS // LANES
NUM_CORES = 2
NUM_SUBCORES = 16
NUM_WORKERS = NUM_CORES * NUM_SUBCORES
SEGS_PER_WORKER = NUM_SEGMENTS // NUM_WORKERS
TILE = 512


def _sc_body(feats_hbm, ids_hbm, w_hbm, b_hbm, bounds_hbm,
             out_sum_hbm, out_max_hbm,
             fbuf, ibuf, wbuf, bbuf, bndbuf, tsum, tmax):
    n_rows = feats_hbm.shape[0]
    wid = lax.axis_index("c") * NUM_SUBCORES + lax.axis_index("s")

    pltpu.sync_copy(w_hbm, wbuf)
    pltpu.sync_copy(b_hbm, bbuf)
    pltpu.sync_copy(bounds_hbm, bndbuf)

    wv = [wbuf[pl.ds(k * LANES, LANES)] for k in range(NCHUNK)]
    bv = bbuf[...]

    zero = jnp.zeros((LANES,), jnp.float32)
    ninf = jnp.full((LANES,), -jnp.inf, jnp.float32)

    def init_row(i, _):
        for k in range(NCHUNK):
            tsum[i, pl.ds(k * LANES, LANES)] = zero
            tmax[i, pl.ds(k * LANES, LANES)] = ninf
        return 0

    lax.fori_loop(0, SEGS_PER_WORKER, init_row, 0)

    bnd = bndbuf[pl.ds(wid, LANES)]
    r0 = bnd[0]
    r1 = bnd[1]
    seg_base = wid * SEGS_PER_WORKER
    r0a = (r0 // 8) * 8
    span = r1 - r0a
    ntiles = jnp.maximum(lax.div(span + TILE - 1, TILE), 0)

    def tile_body(t, _):
        start = r0a + t * TILE
        start_c = jnp.minimum(start, n_rows - TILE)
        pltpu.sync_copy(feats_hbm.at[pl.ds(start_c, TILE)], fbuf)
        pltpu.sync_copy(ids_hbm.at[pl.ds(start_c, TILE)], ibuf.at[pl.ds(0, TILE)])
        lo = jnp.maximum(r0, start) - start_c
        hi = jnp.minimum(r1, start + TILE) - start_c

        def row_body(i, _):
            s = ibuf[pl.ds(i, LANES)][0] - seg_base
            x = [fbuf[i, pl.ds(k * LANES, LANES)] for k in range(NCHUNK)]
            acc = x[0] * wv[0]
            for k in range(1, NCHUNK):
                acc = acc + x[k] * wv[k]
            z = jnp.sum(acc) + bv
            gate = 1.0 / (1.0 + jnp.exp(-z))
            for k in range(NCHUNK):
                sl = pl.ds(k * LANES, LANES)
                tsum[s, sl] = tsum[s, sl] + x[k] * gate
                tmax[s, sl] = jnp.maximum(tmax[s, sl], x[k])
            return 0

        lax.fori_loop(lo, hi, row_body, 0)
        return 0

    lax.fori_loop(0, ntiles, tile_body, 0)

    pltpu.sync_copy(tsum, out_sum_hbm.at[pl.ds(seg_base, SEGS_PER_WORKER)])
    pltpu.sync_copy(tmax, out_max_hbm.at[pl.ds(seg_base, SEGS_PER_WORKER)])


@jax.jit
def _sc_call(feats, ids32, wvec, bvec, bounds):
    mesh = plsc.VectorSubcoreMesh(
        core_axis_name="c", subcore_axis_name="s",
        num_cores=NUM_CORES, num_subcores=NUM_SUBCORES)
    fn = pl.kernel(
        _sc_body,
        out_type=[
            jax.ShapeDtypeStruct((NUM_SEGMENTS, FEATS), jnp.float32),
            jax.ShapeDtypeStruct((NUM_SEGMENTS, FEATS), jnp.float32),
        ],
        mesh=mesh,
        scratch_types=[
            pltpu.VMEM((TILE, FEATS), jnp.float32),
            pltpu.VMEM((TILE + LANES,), jnp.int32),
            pltpu.VMEM((FEATS,), jnp.float32),
            pltpu.VMEM((LANES,), jnp.float32),
            pltpu.VMEM((48,), jnp.int32),
            pltpu.VMEM((SEGS_PER_WORKER, FEATS), jnp.float32),
            pltpu.VMEM((SEGS_PER_WORKER, FEATS), jnp.float32),
        ],
        compiler_params=pltpu.CompilerParams(needs_layout_passes=False),
    )
    return fn(feats, ids32, wvec, bvec, bounds)


def kernel(feats, segment_ids, W, b):
    ids32 = segment_ids.astype(jnp.int32)
    probes = jnp.arange(0, NUM_SEGMENTS + 1, SEGS_PER_WORKER, dtype=jnp.int32)
    bounds = jnp.searchsorted(ids32, probes, side="left").astype(jnp.int32)
    bounds = jnp.pad(bounds, (0, 48 - bounds.shape[0]))
    wvec = W.reshape(FEATS).astype(jnp.float32)
    bvec = jnp.broadcast_to(b.reshape(()), (LANES,)).astype(jnp.float32)
    out_sum, out_max = _sc_call(feats, ids32, wvec, bvec, bounds)
    return jnp.concatenate([out_sum, out_max], axis=1)


# speedup vs baseline: 3.8956x; 1.1613x over previous
import functools

import jax
import jax.numpy as jnp
from jax import lax
from jax.experimental import pallas as pl
from jax.experimental.pallas import tpu as pltpu
from jax.experimental.pallas import tpu_sc as plsc

NUM_SEGMENTS = 512
FEATS = 128
LANES = 16
NCHUNK = FEATS // LANES
NUM_CORES = 2
NUM_SUBCORES = 16
NUM_WORKERS = NUM_CORES * NUM_SUBCORES
SEGS_PER_WORKER = NUM_SEGMENTS // NUM_WORKERS
TILE = 512


def _sc_body(feats_hbm, ids_hbm, w_hbm, b_hbm, bounds_hbm,
             out_sum_hbm, out_max_hbm,
             fbuf, ibuf, wbuf, bbuf, bndbuf, tsum, tmax):
    n_rows = feats_hbm.shape[0]
    wid = lax.axis_index("c") * NUM_SUBCORES + lax.axis_index("s")

    pltpu.sync_copy(w_hbm, wbuf)
    pltpu.sync_copy(b_hbm, bbuf)
    pltpu.sync_copy(bounds_hbm, bndbuf)

    wv = [wbuf[pl.ds(k * LANES, LANES)] for k in range(NCHUNK)]
    bv = bbuf[...]

    zero = jnp.zeros((LANES,), jnp.float32)
    ninf = jnp.full((LANES,), -jnp.inf, jnp.float32)

    def init_row(i, _):
        for k in range(NCHUNK):
            tsum[i, pl.ds(k * LANES, LANES)] = zero
            tmax[i, pl.ds(k * LANES, LANES)] = ninf
        return 0

    lax.fori_loop(0, SEGS_PER_WORKER, init_row, 0)

    bnd = bndbuf[pl.ds(wid, LANES)]
    r0 = bnd[0]
    r1 = bnd[1]
    seg_base = wid * SEGS_PER_WORKER
    r0a = (r0 // 8) * 8
    span = r1 - r0a
    ntiles = jnp.maximum(lax.div(span + TILE - 1, TILE), 0)
    neg_inf = jnp.full((LANES,), -jnp.inf, jnp.float32)

    def tile_body(t, carry):
        start = r0a + t * TILE
        start_c = jnp.minimum(start, n_rows - TILE)
        pltpu.sync_copy(feats_hbm.at[pl.ds(start_c, TILE)], fbuf)
        pltpu.sync_copy(ids_hbm.at[pl.ds(start_c, TILE)], ibuf.at[pl.ds(0, TILE)])
        lo_g = jnp.maximum(r0, start)

        def row_body(i, carry):
            cur, acc, mx = carry
            gg = start_c + i
            valid = (gg >= lo_g) & (gg < r1)
            s = ibuf[pl.ds(i, LANES)][0]
            x = [fbuf[i, pl.ds(k * LANES, LANES)] for k in range(NCHUNK)]
            d = x[0] * wv[0]
            for k in range(1, NCHUNK):
                d = d + x[k] * wv[k]
            z = jnp.sum(d) + bv
            gate = 1.0 / (1.0 + jnp.exp(-z))
            s_eff = jnp.where(valid, s, cur)
            flush = s_eff != cur

            @pl.when(flush & (cur >= 0))
            def _():
                row = cur - seg_base
                for k in range(NCHUNK):
                    sl = pl.ds(k * LANES, LANES)
                    tsum[row, sl] = acc[k]
                    tmax[row, sl] = mx[k]

            keep = jnp.where(flush, 0.0, 1.0).astype(jnp.float32)
            gv = gate * jnp.where(valid, 1.0, 0.0).astype(jnp.float32)
            new_acc = tuple(acc[k] * keep + x[k] * gv for k in range(NCHUNK))
            xm = [jnp.where(valid, x[k], neg_inf) for k in range(NCHUNK)]
            new_mx = tuple(
                jnp.where(flush, xm[k], jnp.maximum(mx[k], xm[k]))
                for k in range(NCHUNK))
            return s_eff, new_acc, new_mx

        return lax.fori_loop(0, TILE, row_body, carry, unroll=4)

    init = (jnp.int32(-1),
            tuple(jnp.zeros((LANES,), jnp.float32) for _ in range(NCHUNK)),
            tuple(neg_inf for _ in range(NCHUNK)))
    cur, acc, mx = lax.fori_loop(0, ntiles, tile_body, init)

    @pl.when(cur >= 0)
    def _():
        row = cur - seg_base
        for k in range(NCHUNK):
            sl = pl.ds(k * LANES, LANES)
            tsum[row, sl] = acc[k]
            tmax[row, sl] = mx[k]

    pltpu.sync_copy(tsum, out_sum_hbm.at[pl.ds(seg_base, SEGS_PER_WORKER)])
    pltpu.sync_copy(tmax, out_max_hbm.at[pl.ds(seg_base, SEGS_PER_WORKER)])


@jax.jit
def _sc_call(feats, ids32, wvec, bvec, bounds):
    mesh = plsc.VectorSubcoreMesh(
        core_axis_name="c", subcore_axis_name="s",
        num_cores=NUM_CORES, num_subcores=NUM_SUBCORES)
    fn = pl.kernel(
        _sc_body,
        out_type=[
            jax.ShapeDtypeStruct((NUM_SEGMENTS, FEATS), jnp.float32),
            jax.ShapeDtypeStruct((NUM_SEGMENTS, FEATS), jnp.float32),
        ],
        mesh=mesh,
        scratch_types=[
            pltpu.VMEM((TILE, FEATS), jnp.float32),
            pltpu.VMEM((TILE + LANES,), jnp.int32),
            pltpu.VMEM((FEATS,), jnp.float32),
            pltpu.VMEM((LANES,), jnp.float32),
            pltpu.VMEM((48,), jnp.int32),
            pltpu.VMEM((SEGS_PER_WORKER, FEATS), jnp.float32),
            pltpu.VMEM((SEGS_PER_WORKER, FEATS), jnp.float32),
        ],
        compiler_params=pltpu.CompilerParams(needs_layout_passes=False),
    )
    return fn(feats, ids32, wvec, bvec, bounds)


def kernel(feats, segment_ids, W, b):
    ids32 = segment_ids.astype(jnp.int32)
    probes = jnp.arange(0, NUM_SEGMENTS + 1, SEGS_PER_WORKER, dtype=jnp.int32)
    bounds = jnp.searchsorted(ids32, probes, side="left").astype(jnp.int32)
    bounds = jnp.pad(bounds, (0, 48 - bounds.shape[0]))
    wvec = W.reshape(FEATS).astype(jnp.float32)
    bvec = jnp.broadcast_to(b.reshape(()), (LANES,)).astype(jnp.float32)
    out_sum, out_max = _sc_call(feats, ids32, wvec, bvec, bounds)
    return jnp.concatenate([out_sum, out_max], axis=1)


# speedup vs baseline: 4.4971x; 1.1544x over previous
import functools

import jax
import jax.numpy as jnp
from jax import lax
from jax.experimental import pallas as pl
from jax.experimental.pallas import tpu as pltpu
from jax.experimental.pallas import tpu_sc as plsc

NUM_SEGMENTS = 512
FEATS = 128
LANES = 16
NCHUNK = FEATS // LANES
NUM_CORES = 2
NUM_SUBCORES = 16
NUM_WORKERS = NUM_CORES * NUM_SUBCORES
SEGS_PER_WORKER = NUM_SEGMENTS // NUM_WORKERS
TILE = 256


def _sc_body(feats_hbm, ids_hbm, w_hbm, b_hbm, bounds_hbm,
             out_sum_hbm, out_max_hbm,
             fbuf0, fbuf1, ibuf0, ibuf1, wbuf, bbuf, bndbuf, tsum, tmax,
             sf0, sf1, si0, si1):
    n_rows = feats_hbm.shape[0]
    wid = lax.axis_index("c") * NUM_SUBCORES + lax.axis_index("s")

    pltpu.sync_copy(w_hbm, wbuf)
    pltpu.sync_copy(b_hbm, bbuf)
    pltpu.sync_copy(bounds_hbm, bndbuf)

    wv = [wbuf[pl.ds(k * LANES, LANES)] for k in range(NCHUNK)]
    bv = bbuf[...]

    zero = jnp.zeros((LANES,), jnp.float32)
    ninf = jnp.full((LANES,), -jnp.inf, jnp.float32)

    def init_row(i, _):
        for k in range(NCHUNK):
            tsum[i, pl.ds(k * LANES, LANES)] = zero
            tmax[i, pl.ds(k * LANES, LANES)] = ninf
        return 0

    lax.fori_loop(0, SEGS_PER_WORKER, init_row, 0)

    bnd = bndbuf[pl.ds(wid, LANES)]
    r0 = bnd[0]
    r1 = bnd[1]
    seg_base = wid * SEGS_PER_WORKER
    r0a = (r0 // 8) * 8
    span = r1 - r0a
    ntiles = jnp.maximum(lax.div(span + TILE - 1, TILE), 0)
    neg_inf = jnp.full((LANES,), -jnp.inf, jnp.float32)

    def tile_start(t):
        start = r0a + t * TILE
        return start, jnp.minimum(start, n_rows - TILE)

    def copies(t, fb, ib, sf, si):
        _, start_c = tile_start(t)
        cf = pltpu.make_async_copy(
            feats_hbm.at[pl.ds(start_c, TILE)], fb, sf)
        ci = pltpu.make_async_copy(
            ids_hbm.at[pl.ds(start_c, TILE)], ib.at[pl.ds(0, TILE)], si)
        return cf, ci

    def issue(t, fb, ib, sf, si):
        cf, ci = copies(t, fb, ib, sf, si)
        cf.start()
        ci.start()

    def wait(t, fb, ib, sf, si):
        cf, ci = copies(t, fb, ib, sf, si)
        cf.wait()
        ci.wait()

    def row_loop(t, fbuf, ibuf, carry):
        start, start_c = tile_start(t)
        lo_g = jnp.maximum(r0, start)

        def row_body(i, carry):
            cur, acc, mx = carry
            gg = start_c + i
            valid = (gg >= lo_g) & (gg < r1)
            s = ibuf[pl.ds(i, LANES)][0]
            x = [fbuf[i, pl.ds(k * LANES, LANES)] for k in range(NCHUNK)]
            d = x[0] * wv[0]
            for k in range(1, NCHUNK):
                d = d + x[k] * wv[k]
            z = jnp.sum(d) + bv
            gate = 1.0 / (1.0 + jnp.exp(-z))
            s_eff = jnp.where(valid, s, cur)
            flush = s_eff != cur

            @pl.when(flush & (cur >= 0))
            def _():
                row = cur - seg_base
                for k in range(NCHUNK):
                    sl = pl.ds(k * LANES, LANES)
                    tsum[row, sl] = acc[k]
                    tmax[row, sl] = mx[k]

            keep = jnp.where(flush, 0.0, 1.0).astype(jnp.float32)
            gv = gate * jnp.where(valid, 1.0, 0.0).astype(jnp.float32)
            new_acc = tuple(acc[k] * keep + x[k] * gv for k in range(NCHUNK))
            xm = [jnp.where(valid, x[k], neg_inf) for k in range(NCHUNK)]
            new_mx = tuple(
                jnp.where(flush, xm[k], jnp.maximum(mx[k], xm[k]))
                for k in range(NCHUNK))
            return s_eff, new_acc, new_mx

        return lax.fori_loop(0, TILE, row_body, carry, unroll=4)

    bufs = ((fbuf0, ibuf0, sf0, si0), (fbuf1, ibuf1, sf1, si1))

    @pl.when(ntiles > 0)
    def _():
        issue(0, *bufs[0])

    def pair_body(h, carry):
        for p in range(2):
            t = 2 * h + p
            nxt = t + 1

            @pl.when(t < ntiles)
            def _():
                wait(t, *bufs[p])

            @pl.when(nxt < ntiles)
            def _():
                issue(nxt, *bufs[1 - p])

            carry = row_loop(t, bufs[p][0], bufs[p][1], carry)
        return carry

    init = (jnp.int32(-1),
            tuple(jnp.zeros((LANES,), jnp.float32) for _ in range(NCHUNK)),
            tuple(neg_inf for _ in range(NCHUNK)))
    npairs = lax.div(ntiles + 1, 2)
    cur, acc, mx = lax.fori_loop(0, npairs, pair_body, init)

    @pl.when(cur >= 0)
    def _():
        row = cur - seg_base
        for k in range(NCHUNK):
            sl = pl.ds(k * LANES, LANES)
            tsum[row, sl] = acc[k]
            tmax[row, sl] = mx[k]

    pltpu.sync_copy(tsum, out_sum_hbm.at[pl.ds(seg_base, SEGS_PER_WORKER)])
    pltpu.sync_copy(tmax, out_max_hbm.at[pl.ds(seg_base, SEGS_PER_WORKER)])


@jax.jit
def _sc_call(feats, ids32, wvec, bvec, bounds):
    mesh = plsc.VectorSubcoreMesh(
        core_axis_name="c", subcore_axis_name="s",
        num_cores=NUM_CORES, num_subcores=NUM_SUBCORES)
    fn = pl.kernel(
        _sc_body,
        out_type=[
            jax.ShapeDtypeStruct((NUM_SEGMENTS, FEATS), jnp.float32),
            jax.ShapeDtypeStruct((NUM_SEGMENTS, FEATS), jnp.float32),
        ],
        mesh=mesh,
        scratch_types=[
            pltpu.VMEM((TILE, FEATS), jnp.float32),
            pltpu.VMEM((TILE, FEATS), jnp.float32),
            pltpu.VMEM((TILE + LANES,), jnp.int32),
            pltpu.VMEM((TILE + LANES,), jnp.int32),
            pltpu.VMEM((FEATS,), jnp.float32),
            pltpu.VMEM((LANES,), jnp.float32),
            pltpu.VMEM((48,), jnp.int32),
            pltpu.VMEM((SEGS_PER_WORKER, FEATS), jnp.float32),
            pltpu.VMEM((SEGS_PER_WORKER, FEATS), jnp.float32),
            pltpu.SemaphoreType.DMA,
            pltpu.SemaphoreType.DMA,
            pltpu.SemaphoreType.DMA,
            pltpu.SemaphoreType.DMA,
        ],
        compiler_params=pltpu.CompilerParams(needs_layout_passes=False),
    )
    return fn(feats, ids32, wvec, bvec, bounds)


def kernel(feats, segment_ids, W, b):
    ids32 = segment_ids.astype(jnp.int32)
    probes = jnp.arange(0, NUM_SEGMENTS + 1, SEGS_PER_WORKER, dtype=jnp.int32)
    bounds = jnp.searchsorted(ids32, probes, side="left").astype(jnp.int32)
    bounds = jnp.pad(bounds, (0, 48 - bounds.shape[0]))
    wvec = W.reshape(FEATS).astype(jnp.float32)
    bvec = jnp.broadcast_to(b.reshape(()), (LANES,)).astype(jnp.float32)
    out_sum, out_max = _sc_call(feats, ids32, wvec, bvec, bounds)
    return jnp.concatenate([out_sum, out_max], axis=1)
